# Initial kernel scaffold; baseline (speedup 1.0000x reference)
import jax
import jax.numpy as jnp
from jax.experimental import pallas as pl


def kernel(weights, table):
    raise NotImplementedError("write your pallas kernel here")


# speedup vs baseline: 16.4918x; 16.4918x over previous
import jax
import jax.numpy as jnp
from jax.experimental import pallas as pl


def _matmul_kernel(w_ref, t_ref, o_ref):
    o_ref[...] = jnp.dot(w_ref[...], t_ref[...],
                         preferred_element_type=jnp.float32)


def kernel(weights, table):
    B, V = weights.shape
    D = table.shape[1]
    return pl.pallas_call(
        _matmul_kernel,
        out_shape=jax.ShapeDtypeStruct((B, D), jnp.float32),
    )(weights, table)
